# baseline (speedup 1.0000x reference)
import functools

import jax
import jax.numpy as jnp
from jax.experimental import pallas as pl
from jax.experimental.pallas import tpu as pltpu


def _bilstm_body(x_ref, wih_ref, whh_ref, b_ref, out_ref, pre_s, out_tm,
                 wih_r, b_r,
                 *, seq_len, nb, hidden, n_chunks):
    T, Nb, H, S = seq_len, nb, hidden, n_chunks
    Tc = T // S
    HH = 2 * H
    s = pl.program_id(0)

    def dmaj(w):
        return jnp.concatenate(
            [w[:, g * HH + d * H: g * HH + (d + 1) * H]
             for d in range(2) for g in range(4)], axis=1)

    @pl.when(s == 0)
    def _prep():
        wih_r[...] = dmaj(wih_ref[...]).astype(jnp.bfloat16)
        b_r[...] = dmaj(b_ref[...])

    xc = x_ref[...].astype(jnp.bfloat16)
    xt = xc.transpose(1, 0, 2).reshape(Tc * Nb, xc.shape[-1])
    pre_s[pl.ds(s * (Tc * Nb), Tc * Nb), :] = jnp.dot(
        xt, wih_r[...], preferred_element_type=jnp.float32) + b_r[...]

    @pl.when(s == S - 1)
    def _recurrence():
        whh = dmaj(whh_ref[...]).astype(jnp.bfloat16)
        whh_f = whh[0:H, 0:4 * H]
        whh_b = whh[H:2 * H, 4 * H:8 * H]

        h_f = jnp.zeros((Nb, H), jnp.float32)
        h_b = jnp.zeros((Nb, H), jnp.float32)
        c_f = jnp.zeros((Nb, H), jnp.float32)
        c_b = jnp.zeros((Nb, H), jnp.float32)

        for t in range(T):
            tb = T - 1 - t
            p_f = pre_s[t * Nb:(t + 1) * Nb, 0:4 * H]
            p_b = pre_s[tb * Nb:(tb + 1) * Nb, 4 * H:8 * H]
            gf = p_f + jnp.dot(h_f.astype(jnp.bfloat16), whh_f,
                               preferred_element_type=jnp.float32)
            gb = p_b + jnp.dot(h_b.astype(jnp.bfloat16), whh_b,
                               preferred_element_type=jnp.float32)
            sf = jax.nn.sigmoid(gf[:, 0:2 * H])
            tf = jnp.tanh(gf[:, 2 * H:3 * H])
            of = jax.nn.sigmoid(gf[:, 3 * H:4 * H])
            sb = jax.nn.sigmoid(gb[:, 0:2 * H])
            tb_g = jnp.tanh(gb[:, 2 * H:3 * H])
            ob = jax.nn.sigmoid(gb[:, 3 * H:4 * H])
            c_f = sf[:, H:2 * H] * c_f + sf[:, 0:H] * tf
            c_b = sb[:, H:2 * H] * c_b + sb[:, 0:H] * tb_g
            h_f = of * jnp.tanh(c_f)
            h_b = ob * jnp.tanh(c_b)
            out_tm[t * Nb:(t + 1) * Nb, 0:H] = h_f
            out_tm[tb * Nb:(tb + 1) * Nb, H:2 * H] = h_b

        out_ref[...] = out_tm[...].reshape(T, Nb, HH).transpose(1, 0, 2)


@jax.jit
def kernel(x, w_ih_fused, w_hh_blk, b_fused):
    N, T, E = x.shape
    H = w_hh_blk.shape[0] // 2
    S = 4
    Tc = T // S

    body = functools.partial(_bilstm_body, seq_len=T, nb=N, hidden=H,
                             n_chunks=S)
    out = pl.pallas_call(
        body,
        out_shape=jax.ShapeDtypeStruct((N, T, 2 * H), x.dtype),
        grid=(S,),
        in_specs=[
            pl.BlockSpec((N, Tc, E), lambda s: (0, s, 0)),
            pl.BlockSpec((E, 8 * H), lambda s: (0, 0)),
            pl.BlockSpec((2 * H, 8 * H), lambda s: (0, 0)),
            pl.BlockSpec((1, 8 * H), lambda s: (0, 0)),
        ],
        out_specs=pl.BlockSpec((N, T, 2 * H), lambda s: (0, 0, 0)),
        scratch_shapes=[
            pltpu.VMEM((T * N, 8 * H), jnp.float32),
            pltpu.VMEM((T * N, 2 * H), jnp.float32),
            pltpu.VMEM((E, 8 * H), jnp.bfloat16),
            pltpu.VMEM((1, 8 * H), jnp.float32),
        ],
        compiler_params=pltpu.CompilerParams(
            dimension_semantics=("arbitrary",)),
    )(x, w_ih_fused, w_hh_blk, b_fused)

    return out


# speedup vs baseline: 1.0023x; 1.0023x over previous
import functools

import jax
import jax.numpy as jnp
from jax.experimental import pallas as pl
from jax.experimental.pallas import tpu as pltpu


def _bilstm_body(x_ref, wih_ref, whh_ref, b_ref, out_ref, pre_s, out_tm,
                 *, seq_len, nb, hidden, n_chunks):
    T, Nb, H, S = seq_len, nb, hidden, n_chunks
    Tc = T // S
    HH = 2 * H
    s = pl.program_id(0)

    def dmaj(w):
        return jnp.concatenate(
            [w[:, g * HH + d * H: g * HH + (d + 1) * H]
             for d in range(2) for g in range(4)], axis=1)

    wih = dmaj(wih_ref[...]).astype(jnp.bfloat16)
    b = dmaj(b_ref[...])

    xc = x_ref[...].astype(jnp.bfloat16)
    xt = xc.transpose(1, 0, 2).reshape(Tc * Nb, xc.shape[-1])
    pre_s[pl.ds(s * (Tc * Nb), Tc * Nb), :] = jnp.dot(
        xt, wih, preferred_element_type=jnp.float32) + b

    @pl.when(s == S - 1)
    def _recurrence():
        whh = dmaj(whh_ref[...]).astype(jnp.bfloat16)
        whh_f = whh[0:H, 0:4 * H]
        whh_b = whh[H:2 * H, 4 * H:8 * H]

        Nh = Nb // 2
        hs = [jnp.zeros((Nh, H), jnp.float32) for _ in range(4)]
        cs = [jnp.zeros((Nh, H), jnp.float32) for _ in range(4)]
        whs = [whh_f, whh_f, whh_b, whh_b]

        for t in range(T):
            tb = T - 1 - t
            rows = [t * Nb, t * Nb + Nh, tb * Nb, tb * Nb + Nh]
            for k in range(4):
                d = k // 2
                p = pre_s[rows[k]:rows[k] + Nh, 4 * H * d:4 * H * (d + 1)]
                g = p + jnp.dot(hs[k].astype(jnp.bfloat16), whs[k],
                                preferred_element_type=jnp.float32)
                sg = jax.nn.sigmoid(g[:, 0:2 * H])
                gg = jnp.tanh(g[:, 2 * H:3 * H])
                og = jax.nn.sigmoid(g[:, 3 * H:4 * H])
                cs[k] = sg[:, H:2 * H] * cs[k] + sg[:, 0:H] * gg
                hs[k] = og * jnp.tanh(cs[k])
                out_tm[rows[k]:rows[k] + Nh, d * H:(d + 1) * H] = hs[k]

        out_ref[...] = out_tm[...].reshape(T, Nb, HH).transpose(1, 0, 2)


@jax.jit
def kernel(x, w_ih_fused, w_hh_blk, b_fused):
    N, T, E = x.shape
    H = w_hh_blk.shape[0] // 2
    S = 4
    Tc = T // S

    body = functools.partial(_bilstm_body, seq_len=T, nb=N, hidden=H,
                             n_chunks=S)
    out = pl.pallas_call(
        body,
        out_shape=jax.ShapeDtypeStruct((N, T, 2 * H), x.dtype),
        grid=(S,),
        in_specs=[
            pl.BlockSpec((N, Tc, E), lambda s: (0, s, 0)),
            pl.BlockSpec((E, 8 * H), lambda s: (0, 0)),
            pl.BlockSpec((2 * H, 8 * H), lambda s: (0, 0)),
            pl.BlockSpec((1, 8 * H), lambda s: (0, 0)),
        ],
        out_specs=pl.BlockSpec((N, T, 2 * H), lambda s: (0, 0, 0)),
        scratch_shapes=[
            pltpu.VMEM((T * N, 8 * H), jnp.float32),
            pltpu.VMEM((T * N, 2 * H), jnp.float32),
        ],
        compiler_params=pltpu.CompilerParams(
            dimension_semantics=("arbitrary",)),
    )(x, w_ih_fused, w_hh_blk, b_fused)

    return out


# speedup vs baseline: 1.0199x; 1.0176x over previous
import functools

import jax
import jax.numpy as jnp
from jax.experimental import pallas as pl
from jax.experimental.pallas import tpu as pltpu


def _bilstm_body(x_ref, wih_ref, whh_ref, b_ref, out_ref, pre_s, out_tm,
                 *, seq_len, nb, hidden, n_chunks):
    T, Nb, H, S = seq_len, nb, hidden, n_chunks
    Tc = T // S
    HH = 2 * H
    s = pl.program_id(0)

    def dmaj(w):
        return jnp.concatenate(
            [w[:, g * HH + d * H: g * HH + (d + 1) * H]
             for d in range(2) for g in range(4)], axis=1)

    wih = dmaj(wih_ref[...]).astype(jnp.bfloat16)
    b = dmaj(b_ref[...])

    Nc = Nb // S
    xc = x_ref[...].astype(jnp.bfloat16)
    xt = xc.transpose(1, 0, 2).reshape(T * Nc, xc.shape[-1])
    res = jnp.dot(xt, wih, preferred_element_type=jnp.float32) + b
    pre_s[:, pl.ds(s, 1), :, :] = res.reshape(T, 1, Nc, 8 * H)

    @pl.when(s == S - 1)
    def _recurrence():
        whh = dmaj(whh_ref[...]).astype(jnp.bfloat16)
        whh_f = whh[0:H, 0:4 * H]
        whh_b = whh[H:2 * H, 4 * H:8 * H]

        h_f = jnp.zeros((Nb, H), jnp.float32)
        h_b = jnp.zeros((Nb, H), jnp.float32)
        c_f = jnp.zeros((Nb, H), jnp.float32)
        c_b = jnp.zeros((Nb, H), jnp.float32)

        for t in range(T):
            tb = T - 1 - t
            p_f = pre_s[t, :, :, 0:4 * H].reshape(Nb, 4 * H)
            p_b = pre_s[tb, :, :, 4 * H:8 * H].reshape(Nb, 4 * H)
            gf = p_f + jnp.dot(h_f.astype(jnp.bfloat16), whh_f,
                               preferred_element_type=jnp.float32)
            gb = p_b + jnp.dot(h_b.astype(jnp.bfloat16), whh_b,
                               preferred_element_type=jnp.float32)
            sf = jax.nn.sigmoid(gf[:, 0:2 * H])
            tf = jnp.tanh(gf[:, 2 * H:3 * H])
            of = jax.nn.sigmoid(gf[:, 3 * H:4 * H])
            sb = jax.nn.sigmoid(gb[:, 0:2 * H])
            tb_g = jnp.tanh(gb[:, 2 * H:3 * H])
            ob = jax.nn.sigmoid(gb[:, 3 * H:4 * H])
            c_f = sf[:, H:2 * H] * c_f + sf[:, 0:H] * tf
            c_b = sb[:, H:2 * H] * c_b + sb[:, 0:H] * tb_g
            h_f = of * jnp.tanh(c_f)
            h_b = ob * jnp.tanh(c_b)
            out_tm[t * Nb:(t + 1) * Nb, 0:H] = h_f
            out_tm[tb * Nb:(tb + 1) * Nb, H:2 * H] = h_b

        out_ref[...] = out_tm[...].reshape(T, Nb, HH).transpose(1, 0, 2)


@jax.jit
def kernel(x, w_ih_fused, w_hh_blk, b_fused):
    N, T, E = x.shape
    H = w_hh_blk.shape[0] // 2
    S = 4
    Tc = T // S

    body = functools.partial(_bilstm_body, seq_len=T, nb=N, hidden=H,
                             n_chunks=S)
    out = pl.pallas_call(
        body,
        out_shape=jax.ShapeDtypeStruct((N, T, 2 * H), x.dtype),
        grid=(S,),
        in_specs=[
            pl.BlockSpec((N // S, T, E), lambda s: (s, 0, 0)),
            pl.BlockSpec((E, 8 * H), lambda s: (0, 0)),
            pl.BlockSpec((2 * H, 8 * H), lambda s: (0, 0)),
            pl.BlockSpec((1, 8 * H), lambda s: (0, 0)),
        ],
        out_specs=pl.BlockSpec((N, T, 2 * H), lambda s: (0, 0, 0)),
        scratch_shapes=[
            pltpu.VMEM((T, S, N // S, 8 * H), jnp.float32),
            pltpu.VMEM((T * N, 2 * H), jnp.float32),
        ],
        compiler_params=pltpu.CompilerParams(
            dimension_semantics=("arbitrary",)),
    )(x, w_ih_fused, w_hh_blk, b_fused)

    return out
